# Initial kernel scaffold; baseline (speedup 1.0000x reference)
import jax
import jax.numpy as jnp
from jax.experimental import pallas as pl


def kernel(x_nchw, w_oihw, conv_bias, bn_gamma, bn_beta):
    raise NotImplementedError("write your pallas kernel here")


# speedup vs baseline: 1.4829x; 1.4829x over previous
import jax
import jax.numpy as jnp
from jax.experimental import pallas as pl
from jax.experimental.pallas import tpu as pltpu

_PARITIES = ((0, 0), (0, 1), (1, 0), (1, 1))


def _patches(xs, a, b, hs, ws, cin):
    cols = []
    for t in (0, 1):
        for u in (0, 1):
            win = xs[a + t:a + t + hs, b + u:b + u + ws, :]
            cols.append(win.reshape(hs * ws, cin))
    return jnp.concatenate(cols, axis=1)


def _stats_kernel(xp_ref, w_ref, psum_ref, psumsq_ref):
    _, hp, wp, cin = xp_ref.shape
    hs, ws = hp - 2, wp - 2
    xs = xp_ref[0]
    s = jnp.zeros((1, w_ref.shape[2]), jnp.float32)
    s2 = jnp.zeros((1, w_ref.shape[2]), jnp.float32)
    for p, (a, b) in enumerate(_PARITIES):
        patch = _patches(xs, a, b, hs, ws, cin)
        acc = jnp.dot(patch, w_ref[p], preferred_element_type=jnp.float32)
        s = s + jnp.sum(acc, axis=0, keepdims=True)
        s2 = s2 + jnp.sum(acc * acc, axis=0, keepdims=True)
    psum_ref[0] = s
    psumsq_ref[0] = s2


def _apply_kernel(xp_ref, w_ref, scale_ref, shift_ref, o_ref):
    _, hp, wp, cin = xp_ref.shape
    hs, ws = hp - 2, wp - 2
    cout = w_ref.shape[2]
    xs = xp_ref[0]
    zs = []
    for p, (a, b) in enumerate(_PARITIES):
        patch = _patches(xs, a, b, hs, ws, cin)
        acc = jnp.dot(patch, w_ref[p], preferred_element_type=jnp.float32)
        z = jnp.maximum(acc * scale_ref[...] + shift_ref[...], 0.0)
        zs.append(z.reshape(hs, ws, 1, cout))
    r0 = jnp.concatenate([zs[0], zs[1]], axis=2).reshape(hs, 1, 2 * ws, cout)
    r1 = jnp.concatenate([zs[2], zs[3]], axis=2).reshape(hs, 1, 2 * ws, cout)
    zz = jnp.concatenate([r0, r1], axis=1).reshape(4 * hs * ws, cout)
    o_ref[0] = zz.T


def _conv_up_bn_relu(x_nchw, w_oihw, conv_bias, bn_gamma, bn_beta,
                     *, eps=1e-5):
    n, cin, h_in, w_in = x_nchw.shape
    cout = w_oihw.shape[0]
    h, w = 2 * h_in, 2 * w_in
    hw = h * w

    x = jnp.transpose(x_nchw, (0, 2, 3, 1)).astype(jnp.bfloat16)
    xp = jnp.pad(x, ((0, 0), (1, 1), (1, 1), (0, 0)))

    w9 = jnp.transpose(w_oihw, (2, 3, 1, 0)).astype(jnp.float32)
    comb = jnp.array([[[1., 0., 0.], [0., 1., 1.]],
                      [[1., 1., 0.], [0., 0., 1.]]], jnp.float32)
    wf = jnp.einsum('atk,bul,klio->abtuio', comb, comb, w9)
    wf = wf.reshape(4, 4 * cin, cout).astype(jnp.bfloat16)

    kb = 4 * cin

    psum, psumsq = pl.pallas_call(
        _stats_kernel,
        out_shape=(
            jax.ShapeDtypeStruct((n, 1, cout), jnp.float32),
            jax.ShapeDtypeStruct((n, 1, cout), jnp.float32),
        ),
        grid=(n,),
        in_specs=[
            pl.BlockSpec((1, h_in + 2, w_in + 2, cin), lambda i: (i, 0, 0, 0)),
            pl.BlockSpec((4, kb, cout), lambda i: (0, 0, 0)),
        ],
        out_specs=(
            pl.BlockSpec((1, 1, cout), lambda i: (i, 0, 0)),
            pl.BlockSpec((1, 1, cout), lambda i: (i, 0, 0)),
        ),
        compiler_params=pltpu.CompilerParams(
            dimension_semantics=("parallel",)),
    )(xp, wf)

    count = jnp.float32(n * hw)
    eacc = jnp.sum(psum[:, 0, :], axis=0) / count
    eacc2 = jnp.sum(psumsq[:, 0, :], axis=0) / count
    bias = conv_bias.astype(jnp.float32)
    mean = eacc + bias
    ex2 = eacc2 + (2.0 * bias) * eacc + bias * bias
    var = jnp.maximum(ex2 - mean * mean, 0.0)
    scale = bn_gamma.astype(jnp.float32) * jax.lax.rsqrt(var + eps)
    shift2 = bn_beta.astype(jnp.float32) - eacc * scale

    out = pl.pallas_call(
        _apply_kernel,
        out_shape=jax.ShapeDtypeStruct((n, cout, hw), x_nchw.dtype),
        grid=(n,),
        in_specs=[
            pl.BlockSpec((1, h_in + 2, w_in + 2, cin), lambda i: (i, 0, 0, 0)),
            pl.BlockSpec((4, kb, cout), lambda i: (0, 0, 0)),
            pl.BlockSpec((1, cout), lambda i: (0, 0)),
            pl.BlockSpec((1, cout), lambda i: (0, 0)),
        ],
        out_specs=pl.BlockSpec((1, cout, hw), lambda i: (i, 0, 0)),
        compiler_params=pltpu.CompilerParams(
            dimension_semantics=("parallel",)),
    )(xp, wf, scale.reshape(1, cout), shift2.reshape(1, cout))

    return out.reshape(n, cout, h, w)


def kernel(x_nchw, w_oihw, conv_bias, bn_gamma, bn_beta):
    return _conv_up_bn_relu(x_nchw, w_oihw, conv_bias, bn_gamma, bn_beta,
                            eps=1e-5)
